# baseline (speedup 1.0000x reference)
import functools

import jax
import jax.numpy as jnp
from jax import lax
from jax.experimental import pallas as pl
from jax.experimental.pallas import tpu as pltpu
from jax.experimental.pallas import tpu_sc as plsc

D = 64
NC = 2
NS = 16
NW = NC * NS
BB = 128
NBUF = 2


def _sc_gather(xt, table, n_seq, n_batch):
    n_bblk = n_batch // BB
    assert n_bblk == NW
    mesh = plsc.VectorSubcoreMesh(core_axis_name="c", subcore_axis_name="s")

    @functools.partial(
        pl.kernel,
        mesh=mesh,
        out_type=jax.ShapeDtypeStruct((n_seq, D // 8, n_bblk, 8, BB), jnp.float32),
        compiler_params=pltpu.CompilerParams(
            use_tc_tiling_on_sc=False, needs_layout_passes=False
        ),
        scratch_types=[
            pltpu.VMEM((NBUF * BB,), jnp.int32),
            pltpu.VMEM((NBUF * BB, D), jnp.bfloat16),
            pltpu.VMEM((NBUF * D, BB + 1), jnp.float32),
            pltpu.SemaphoreType.DMA,
            pltpu.SemaphoreType.DMA,
            pltpu.SemaphoreType.DMA,
            pltpu.SemaphoreType.DMA,
            pltpu.SemaphoreType.DMA,
            pltpu.SemaphoreType.DMA,
        ],
    )
    def k(xt_hbm, table_hbm, out_hbm, idx_v, rows_v, stage_v, g0, g1, w0, w1, i0, i1):
        gsem = (g0, g1)
        wsem = (w0, w1)
        isem = (i0, i1)
        wid = lax.axis_index("s") * NC + lax.axis_index("c")
        lane16 = lax.iota(jnp.int32, 16)
        lane2 = lane16 * 2

        def idx_src(s):
            return xt_hbm.at[s, pl.ds(wid * BB, BB)]

        def idx_dst(slot):
            return idx_v.at[pl.ds(slot * BB, BB)]

        def gather_copy(slot):
            return pltpu.make_async_copy(
                table_hbm.at[idx_v.at[pl.ds(slot * BB, BB)]],
                rows_v.at[pl.ds(slot * BB, BB)],
                gsem[slot],
            )

        def out_copies(s, slot):
            return [
                pltpu.make_async_copy(
                    stage_v.at[pl.ds(slot * D + 8 * j, 8), pl.ds(0, BB)],
                    out_hbm.at[s, j, wid],
                    wsem[slot],
                )
                for j in range(D // 8)
            ]

        def transpose(slot):
            row0 = slot * BB
            srow0 = slot * D

            def body(t8, carry):
                t0 = t8 * 8
                for dt in range(8):
                    tcol = jnp.full((16,), t0 + dt, jnp.int32)
                    for h in range(D // 32):
                        raw = rows_v[row0 + t0 + dt, pl.ds(32 * h, 32)]
                        ri = plsc.bitcast(raw, jnp.int32)
                        even = plsc.bitcast(ri << 16, jnp.float32)
                        odd = plsc.bitcast(
                            ri & jnp.int32(-65536), jnp.float32
                        )
                        plsc.store_scatter(
                            stage_v, [lane2 + (srow0 + 32 * h), tcol], even
                        )
                        plsc.store_scatter(
                            stage_v, [lane2 + (srow0 + 32 * h + 1), tcol], odd
                        )
                return carry

            lax.fori_loop(0, BB // 8, body, 0)

        pltpu.sync_copy(idx_src(0), idx_dst(0))
        gather_copy(0).start()
        pltpu.async_copy(idx_src(1), idx_dst(1), isem[1])

        def stage(s, slot):
            nslot = 1 - slot

            @pl.when(s + 1 < n_seq)
            def _():
                pltpu.make_async_copy(idx_src(s + 1), idx_dst(nslot), isem[nslot]).wait()
                gather_copy(nslot).start()

            gather_copy(slot).wait()

            @pl.when(s + 2 < n_seq)
            def _():
                pltpu.async_copy(idx_src(s + 2), idx_dst(slot), isem[slot])

            @pl.when(s >= NBUF)
            def _():
                for c in out_copies(s - NBUF, slot):
                    c.wait()

            transpose(slot)

            for c in out_copies(s, slot):
                c.start()

        def outer(p, carry):
            stage(NBUF * p, 0)
            stage(NBUF * p + 1, 1)
            return carry

        lax.fori_loop(0, n_seq // NBUF, outer, 0)

        for t in (n_seq - 2, n_seq - 1):
            for c in out_copies(t, t % NBUF):
                c.wait()

    return k(xt, table)


def kernel(x, table):
    b, s = x.shape
    xt = x.T.astype(jnp.int32)
    table_bf = table.astype(jnp.bfloat16)
    out5 = _sc_gather(xt, table_bf, s, b)
    return out5.transpose(2, 4, 0, 1, 3).reshape(b, s, D)


# speedup vs baseline: 1.2237x; 1.2237x over previous
import functools

import jax
import jax.numpy as jnp
from jax import lax
from jax.experimental import pallas as pl
from jax.experimental.pallas import tpu as pltpu
from jax.experimental.pallas import tpu_sc as plsc

D = 64
NC = 2
NS = 16
NW = NC * NS
BB = 128
NBUF = 2


def _sc_gather(xt, table, n_seq, n_batch):
    n_bblk = n_batch // BB
    assert n_bblk == NW
    mesh = plsc.VectorSubcoreMesh(core_axis_name="c", subcore_axis_name="s")

    @functools.partial(
        pl.kernel,
        mesh=mesh,
        out_type=jax.ShapeDtypeStruct((n_seq, D // 8, n_bblk, 8, BB), jnp.float32),
        compiler_params=pltpu.CompilerParams(
            use_tc_tiling_on_sc=False, needs_layout_passes=False
        ),
        scratch_types=[
            pltpu.VMEM((NBUF * BB,), jnp.int32),
            pltpu.VMEM((NBUF * BB, 2 * D), jnp.float32),
            pltpu.VMEM((NBUF * D, BB + 1), jnp.float32),
            pltpu.SemaphoreType.DMA,
            pltpu.SemaphoreType.DMA,
            pltpu.SemaphoreType.DMA,
            pltpu.SemaphoreType.DMA,
            pltpu.SemaphoreType.DMA,
            pltpu.SemaphoreType.DMA,
        ],
    )
    def k(xt_hbm, table_hbm, out_hbm, idx_v, rows_v, stage_v, g0, g1, w0, w1, i0, i1):
        gsem = (g0, g1)
        wsem = (w0, w1)
        isem = (i0, i1)
        wid = lax.axis_index("s") * NC + lax.axis_index("c")
        lane16 = lax.iota(jnp.int32, 16)

        def idx_src(s):
            return xt_hbm.at[s, pl.ds(wid * BB, BB)]

        def idx_dst(slot):
            return idx_v.at[pl.ds(slot * BB, BB)]

        def gather_copy(slot):
            return pltpu.make_async_copy(
                table_hbm.at[idx_v.at[pl.ds(slot * BB, BB)]],
                rows_v.at[pl.ds(slot * BB, BB)],
                gsem[slot],
            )

        def out_copies(s, slot):
            return [
                pltpu.make_async_copy(
                    stage_v.at[pl.ds(slot * D + 8 * j, 8), pl.ds(0, BB)],
                    out_hbm.at[s, j, wid],
                    wsem[slot],
                )
                for j in range(D // 8)
            ]

        def transpose(slot):
            row0 = slot * BB
            srow0 = slot * D

            def body(t8, carry):
                t0 = t8 * 8
                for dt in range(8):
                    for f in range(D // 16):
                        vals = rows_v[row0 + t0 + dt, pl.ds(16 * f, 16)]
                        plsc.store_scatter(
                            stage_v,
                            [
                                lane16 + (srow0 + 16 * f),
                                jnp.full((16,), t0 + dt, jnp.int32),
                            ],
                            vals,
                        )
                return carry

            lax.fori_loop(0, BB // 8, body, 0)

        pltpu.sync_copy(idx_src(0), idx_dst(0))
        gather_copy(0).start()
        pltpu.async_copy(idx_src(1), idx_dst(1), isem[1])

        def stage(s, slot):
            nslot = 1 - slot

            @pl.when(s + 1 < n_seq)
            def _():
                pltpu.make_async_copy(idx_src(s + 1), idx_dst(nslot), isem[nslot]).wait()
                gather_copy(nslot).start()

            gather_copy(slot).wait()

            @pl.when(s + 2 < n_seq)
            def _():
                pltpu.async_copy(idx_src(s + 2), idx_dst(slot), isem[slot])

            @pl.when(s >= NBUF)
            def _():
                for c in out_copies(s - NBUF, slot):
                    c.wait()

            transpose(slot)

            for c in out_copies(s, slot):
                c.start()

        def outer(p, carry):
            stage(NBUF * p, 0)
            stage(NBUF * p + 1, 1)
            return carry

        lax.fori_loop(0, n_seq // NBUF, outer, 0)

        for t in (n_seq - 2, n_seq - 1):
            for c in out_copies(t, t % NBUF):
                c.wait()

    return k(xt, table)


def kernel(x, table):
    b, s = x.shape
    xt = x.T.astype(jnp.int32)
    table128 = jnp.pad(table, ((0, 0), (0, D)))
    out5 = _sc_gather(xt, table128, s, b)
    return out5.transpose(2, 4, 0, 1, 3).reshape(b, s, D)
